# baseline (speedup 1.0000x reference)
import functools

import jax
import jax.numpy as jnp
from jax import lax
from jax.experimental import pallas as pl
from jax.experimental.pallas import tpu as pltpu
from jax.experimental.pallas import tpu_sc as plsc

N = 10000
E = 320000
HID = 128
NCLS = 10
L = 4

SC_CORES = 2
SC_SUBCORES = 16
NW = SC_CORES * SC_SUBCORES
CH = 128
CPW = 80
SLAB = CPW // 2
EPAD = NW * CPW * CH
NACC = N + 16
WB = 624
WB_TAIL_BASE = WB * SC_SUBCORES
WB_TAIL = N - WB_TAIL_BASE


def _sc_agg_body(h_hbm, src_hbm, dst_hbm, zeros_hbm, out_hbm,
                 acc_sh, src_sl, dst_sl, rows0, rows1, sem0, sem1):
    c = lax.axis_index("c")
    s = lax.axis_index("s")
    w = c * SC_SUBCORES + s
    chunk0 = w * CPW

    def _gather(i, buf, sem):
        pltpu.async_copy(h_hbm.at[src_sl.at[i, pl.ds(0, CH // 2)]],
                         buf.at[pl.ds(0, CH // 2)], sem)
        pltpu.async_copy(h_hbm.at[src_sl.at[i, pl.ds(CH // 2, CH // 2)]],
                         buf.at[pl.ds(CH // 2, CH // 2)], sem)

    pltpu.sync_copy(zeros_hbm, acc_sh.at[pl.ds(s * WB, WB)])

    @pl.when(s == 0)
    def _():
        pltpu.sync_copy(zeros_hbm.at[pl.ds(0, WB_TAIL + 16)],
                        acc_sh.at[pl.ds(WB_TAIL_BASE, WB_TAIL + 16)])

    plsc.subcore_barrier()

    @pl.loop(0, 2)
    def _(p):
        pltpu.sync_copy(src_hbm.at[pl.ds(chunk0 + p * SLAB, SLAB)], src_sl)
        pltpu.sync_copy(dst_hbm.at[pl.ds(chunk0 + p * SLAB, SLAB)], dst_sl)
        _gather(0, rows0, sem0)
        _gather(1, rows1, sem1)

        @pl.loop(0, SLAB, step=2)
        def _(i):
            pltpu.make_async_copy(h_hbm.at[src_sl.at[i]], rows0, sem0).wait()
            pltpu.sync_copy(rows0, acc_sh.at[dst_sl.at[i]], add=True)

            @pl.when(i + 2 < SLAB)
            def _():
                _gather(i + 2, rows0, sem0)

            pltpu.make_async_copy(h_hbm.at[src_sl.at[i + 1]], rows1, sem1).wait()
            pltpu.sync_copy(rows1, acc_sh.at[dst_sl.at[i + 1]], add=True)

            @pl.when(i + 3 < SLAB)
            def _():
                _gather(i + 3, rows1, sem1)

    plsc.subcore_barrier()
    pltpu.sync_copy(acc_sh.at[pl.ds(s * WB, WB)],
                    out_hbm.at[c, pl.ds(s * WB, WB)])

    @pl.when(s == 0)
    def _():
        pltpu.sync_copy(acc_sh.at[pl.ds(WB_TAIL_BASE, WB_TAIL)],
                        out_hbm.at[c, pl.ds(WB_TAIL_BASE, WB_TAIL)])


@functools.cache
def _sc_aggregate_fn():
    return pl.kernel(
        _sc_agg_body,
        out_type=jax.ShapeDtypeStruct((SC_CORES, N, HID), jnp.float32),
        mesh=plsc.VectorSubcoreMesh(core_axis_name="c", subcore_axis_name="s",
                                    num_cores=SC_CORES,
                                    num_subcores=SC_SUBCORES),
        scratch_types=[
            pltpu.VMEM_SHARED((NACC, HID), jnp.float32),
            pltpu.VMEM((SLAB, CH), jnp.int32),
            pltpu.VMEM((SLAB, CH), jnp.int32),
            pltpu.VMEM((CH, HID), jnp.float32),
            pltpu.VMEM((CH, HID), jnp.float32),
            pltpu.SemaphoreType.DMA,
            pltpu.SemaphoreType.DMA,
        ],
    )


def _enc_body(h_ref, w_ref, b_ref, out_ref, cs_ref):
    z = jnp.dot(h_ref[...], w_ref[...],
                preferred_element_type=jnp.float32) + b_ref[...]
    out_ref[...] = z
    cs_ref[...] = jnp.sum(z, axis=0, keepdims=True)


_encoder = pl.pallas_call(
    _enc_body,
    out_shape=[
        jax.ShapeDtypeStruct((N, HID), jnp.float32),
        jax.ShapeDtypeStruct((1, HID), jnp.float32),
    ],
)


def _bn_cols(z, g, b):
    mu = jnp.mean(z, axis=0, keepdims=True)
    var = jnp.mean((z - mu) * (z - mu), axis=0, keepdims=True)
    return g * (z - mu) * lax.rsqrt(var + 1e-5) + b


def _layer_body(h_ref, agg_ref, w1_ref, b1_ref, g1_ref, be1_ref,
                w2_ref, b2_ref, gl_ref, bel_ref, out_ref, cs_ref):
    h = h_ref[...]
    hh = h + agg_ref[0] + agg_ref[1]
    z = jnp.dot(hh, w1_ref[...], preferred_element_type=jnp.float32) + b1_ref[...]
    z = jnp.maximum(_bn_cols(z, g1_ref[...], be1_ref[...]), 0.0)
    z = jnp.dot(z, w2_ref[...], preferred_element_type=jnp.float32) + b2_ref[...]
    z = jnp.maximum(_bn_cols(z, gl_ref[...], bel_ref[...]), 0.0)
    hout = h + z
    out_ref[...] = hout
    cs_ref[...] = jnp.sum(hout, axis=0, keepdims=True)


_gin_layer = pl.pallas_call(
    _layer_body,
    out_shape=[
        jax.ShapeDtypeStruct((N, HID), jnp.float32),
        jax.ShapeDtypeStruct((1, HID), jnp.float32),
    ],
)


def _last_body(h_ref, agg_ref, w1_ref, b1_ref, g1_ref, be1_ref,
               w2_ref, b2_ref, gl_ref, bel_ref, cs_ref, wp_ref, bp_ref,
               out_ref):
    h = h_ref[...]
    hh = h + agg_ref[0] + agg_ref[1]
    z = jnp.dot(hh, w1_ref[...], preferred_element_type=jnp.float32) + b1_ref[...]
    z = jnp.maximum(_bn_cols(z, g1_ref[...], be1_ref[...]), 0.0)
    z = jnp.dot(z, w2_ref[...], preferred_element_type=jnp.float32) + b2_ref[...]
    z = jnp.maximum(_bn_cols(z, gl_ref[...], bel_ref[...]), 0.0)
    cs_last = jnp.sum(h + z, axis=0, keepdims=True)
    acc = jnp.dot(cs_last * (1.0 / N), wp_ref[L],
                  preferred_element_type=jnp.float32) + bp_ref[L:L + 1, :]
    for i in range(L):
        acc = acc + jnp.dot(cs_ref[i:i + 1, :] * (1.0 / N), wp_ref[i],
                            preferred_element_type=jnp.float32)
        acc = acc + bp_ref[i:i + 1, :]
    out_ref[...] = acc


_gin_last = pl.pallas_call(
    _last_body,
    out_shape=jax.ShapeDtypeStruct((1, NCLS), jnp.float32),
)


def kernel(h, edge_index, e, W_enc, b_enc, W1, b1, g1, be1, W2, b2,
           gL, beL, Wp, bp):
    del e
    npad = EPAD - E
    fill = (jnp.arange(npad, dtype=jnp.int32) % 16)
    src = jnp.concatenate([edge_index[0].astype(jnp.int32), fill])
    dst = jnp.concatenate([edge_index[1].astype(jnp.int32), N + fill])
    src = src.reshape(NW * CPW, CH)
    dst = dst.reshape(NW * CPW, CH)
    zeros = jnp.zeros((WB, HID), jnp.float32)

    hcur, cs = _encoder(h, W_enc, b_enc.reshape(1, HID))
    colsums = [cs]
    for i in range(L - 1):
        agg = _sc_aggregate_fn()(hcur, src, dst, zeros)
        hcur, cs = _gin_layer(
            hcur, agg, W1[i], b1[i].reshape(1, HID), g1[i].reshape(1, HID),
            be1[i].reshape(1, HID), W2[i], b2[i].reshape(1, HID),
            gL[i].reshape(1, HID), beL[i].reshape(1, HID))
        colsums.append(cs)
    agg = _sc_aggregate_fn()(hcur, src, dst, zeros)
    i = L - 1
    cs_all = jnp.concatenate(colsums, axis=0)
    return _gin_last(
        hcur, agg, W1[i], b1[i].reshape(1, HID), g1[i].reshape(1, HID),
        be1[i].reshape(1, HID), W2[i], b2[i].reshape(1, HID),
        gL[i].reshape(1, HID), beL[i].reshape(1, HID), cs_all, Wp, bp)


# speedup vs baseline: 1.0655x; 1.0655x over previous
import functools

import jax
import jax.numpy as jnp
from jax import lax
from jax.experimental import pallas as pl
from jax.experimental.pallas import tpu as pltpu
from jax.experimental.pallas import tpu_sc as plsc

N = 10000
E = 320000
HID = 128
NCLS = 10
L = 4

SC_CORES = 2
SC_SUBCORES = 16
NW = SC_CORES * SC_SUBCORES
CH = 128
CPW = 80
SLAB = CPW // 2
EPAD = NW * CPW * CH
NACC = N + 16
WB = 624
WB_TAIL_BASE = WB * SC_SUBCORES
WB_TAIL = N - WB_TAIL_BASE


def _sc_agg_body(h_hbm, src_hbm, dst_hbm, zeros_hbm, out_hbm,
                 acc_sh, src_sl, dst_sl, rows0, rows1, sem0, sem1):
    c = lax.axis_index("c")
    s = lax.axis_index("s")
    w = c * SC_SUBCORES + s
    chunk0 = w * CPW

    def _gather(i, buf, sem):
        pltpu.async_copy(h_hbm.at[src_sl.at[i, pl.ds(0, CH // 2)]],
                         buf.at[pl.ds(0, CH // 2)], sem)
        pltpu.async_copy(h_hbm.at[src_sl.at[i, pl.ds(CH // 2, CH // 2)]],
                         buf.at[pl.ds(CH // 2, CH // 2)], sem)

    pltpu.sync_copy(zeros_hbm, acc_sh.at[pl.ds(s * WB, WB)])

    @pl.when(s == 0)
    def _():
        pltpu.sync_copy(zeros_hbm.at[pl.ds(0, WB_TAIL + 16)],
                        acc_sh.at[pl.ds(WB_TAIL_BASE, WB_TAIL + 16)])

    plsc.subcore_barrier()

    @pl.loop(0, 2)
    def _(p):
        pltpu.sync_copy(src_hbm.at[pl.ds(chunk0 + p * SLAB, SLAB)], src_sl)
        pltpu.sync_copy(dst_hbm.at[pl.ds(chunk0 + p * SLAB, SLAB)], dst_sl)
        _gather(0, rows0, sem0)
        _gather(1, rows1, sem1)

        @pl.loop(0, SLAB, step=2)
        def _(i):
            pltpu.make_async_copy(h_hbm.at[src_sl.at[i]], rows0, sem0).wait()
            pltpu.sync_copy(rows0, acc_sh.at[dst_sl.at[i]], add=True)

            @pl.when(i + 2 < SLAB)
            def _():
                _gather(i + 2, rows0, sem0)

            pltpu.make_async_copy(h_hbm.at[src_sl.at[i + 1]], rows1, sem1).wait()
            pltpu.sync_copy(rows1, acc_sh.at[dst_sl.at[i + 1]], add=True)

            @pl.when(i + 3 < SLAB)
            def _():
                _gather(i + 3, rows1, sem1)

    plsc.subcore_barrier()
    pltpu.sync_copy(acc_sh.at[pl.ds(s * WB, WB)],
                    out_hbm.at[c, pl.ds(s * WB, WB)])

    @pl.when(s == 0)
    def _():
        pltpu.sync_copy(acc_sh.at[pl.ds(WB_TAIL_BASE, WB_TAIL)],
                        out_hbm.at[c, pl.ds(WB_TAIL_BASE, WB_TAIL)])


@functools.cache
def _sc_aggregate_fn():
    return pl.kernel(
        _sc_agg_body,
        out_type=jax.ShapeDtypeStruct((SC_CORES, N, HID), jnp.float32),
        mesh=plsc.VectorSubcoreMesh(core_axis_name="c", subcore_axis_name="s",
                                    num_cores=SC_CORES,
                                    num_subcores=SC_SUBCORES),
        scratch_types=[
            pltpu.VMEM_SHARED((NACC, HID), jnp.float32),
            pltpu.VMEM((SLAB, CH), jnp.int32),
            pltpu.VMEM((SLAB, CH), jnp.int32),
            pltpu.VMEM((CH, HID), jnp.float32),
            pltpu.VMEM((CH, HID), jnp.float32),
            pltpu.SemaphoreType.DMA,
            pltpu.SemaphoreType.DMA,
        ],
    )


def _enc_body(h_ref, w_ref, b_ref, out_ref, cs_ref):
    z = jnp.dot(h_ref[...], w_ref[...],
                preferred_element_type=jnp.float32) + b_ref[...]
    out_ref[...] = z
    cs_ref[...] = jnp.sum(z, axis=0, keepdims=True)


_encoder = pl.pallas_call(
    _enc_body,
    out_shape=[
        jax.ShapeDtypeStruct((N, HID), jnp.float32),
        jax.ShapeDtypeStruct((1, HID), jnp.float32),
    ],
)


def _bn_cols(z, g, b):
    mu = jnp.mean(z, axis=0, keepdims=True)
    var = jnp.mean((z - mu) * (z - mu), axis=0, keepdims=True)
    return g * (z - mu) * lax.rsqrt(var + 1e-5) + b


def _layer_body(h_ref, agg_ref, w1_ref, b1_ref, g1_ref, be1_ref,
                w2_ref, b2_ref, gl_ref, bel_ref, out_ref, cs_ref):
    h = h_ref[...]
    hh = h + agg_ref[0] + agg_ref[1]
    z = jnp.dot(hh, w1_ref[...], preferred_element_type=jnp.float32) + b1_ref[...]
    z = jnp.maximum(_bn_cols(z, g1_ref[...], be1_ref[...]), 0.0)
    z = jnp.dot(z, w2_ref[...], preferred_element_type=jnp.float32) + b2_ref[...]
    z = jnp.maximum(_bn_cols(z, gl_ref[...], bel_ref[...]), 0.0)
    hout = h + z
    out_ref[...] = hout
    cs_ref[...] = jnp.sum(hout, axis=0, keepdims=True)


_gin_layer = pl.pallas_call(
    _layer_body,
    out_shape=[
        jax.ShapeDtypeStruct((N, HID), jnp.float32),
        jax.ShapeDtypeStruct((1, HID), jnp.float32),
    ],
)


def _readout_body(cs_ref, wp_ref, bp_ref, out_ref):
    acc = jnp.zeros((1, NCLS), jnp.float32)
    for i in range(L + 1):
        acc = acc + jnp.dot(cs_ref[i:i + 1, :] * (1.0 / N), wp_ref[i],
                            preferred_element_type=jnp.float32)
        acc = acc + bp_ref[i:i + 1, :]
    out_ref[...] = acc


_readout = pl.pallas_call(
    _readout_body,
    out_shape=jax.ShapeDtypeStruct((1, NCLS), jnp.float32),
)


def kernel(h, edge_index, e, W_enc, b_enc, W1, b1, g1, be1, W2, b2,
           gL, beL, Wp, bp):
    del e
    npad = EPAD - E
    fill = (jnp.arange(npad, dtype=jnp.int32) % 16)
    src = jnp.concatenate([edge_index[0].astype(jnp.int32), fill])
    dst = jnp.concatenate([edge_index[1].astype(jnp.int32), N + fill])
    src = src.reshape(NW * CPW, CH)
    dst = dst.reshape(NW * CPW, CH)
    zeros = jnp.zeros((WB, HID), jnp.float32)

    hcur, cs = _encoder(h, W_enc, b_enc.reshape(1, HID))
    colsums = [cs]
    for i in range(L):
        agg = _sc_aggregate_fn()(hcur, src, dst, zeros)
        hcur, cs = _gin_layer(
            hcur, agg, W1[i], b1[i].reshape(1, HID), g1[i].reshape(1, HID),
            be1[i].reshape(1, HID), W2[i], b2[i].reshape(1, HID),
            gL[i].reshape(1, HID), beL[i].reshape(1, HID))
        colsums.append(cs)
    cs_all = jnp.concatenate(colsums, axis=0)
    return _readout(cs_all, Wp, bp)
